# Initial kernel scaffold; baseline (speedup 1.0000x reference)
import jax
import jax.numpy as jnp
from jax.experimental import pallas as pl


def kernel(x, emb_table, W, b):
    raise NotImplementedError("write your pallas kernel here")


# speedup vs baseline: 3.9197x; 3.9197x over previous
import functools

import jax
import jax.numpy as jnp
from jax import lax
from jax.experimental import pallas as pl
from jax.experimental.pallas import tpu as pltpu
from jax.experimental.pallas import tpu_sc as plsc

_VOCAB = 100000
_DIM = 300
_CLASSES = 6
_BATCH = 4096
_SEQ = 50

_CP = 16
_NC = 2
_NS = 16
_NW = _NC * _NS
_BPW = _BATCH // _NW
_IPW = _BPW * _SEQ
_ROWS_BLK = 2000


def _proj_body(emb_ref, w_ref, out_ref):
    out_ref[...] = jnp.dot(
        emb_ref[...], w_ref[...], preferred_element_type=jnp.float32
    )


def _project_table(emb_table, w_pad):
    return pl.pallas_call(
        _proj_body,
        grid=(_VOCAB // _ROWS_BLK,),
        in_specs=[
            pl.BlockSpec((_ROWS_BLK, _DIM), lambda i: (i, 0)),
            pl.BlockSpec((_DIM, _CP), lambda i: (0, 0)),
        ],
        out_specs=pl.BlockSpec((_ROWS_BLK, _CP), lambda i: (i, 0)),
        out_shape=jax.ShapeDtypeStruct((_VOCAB, _CP), jnp.float32),
    )(emb_table, w_pad)


_sc_mesh = plsc.VectorSubcoreMesh(core_axis_name="c", subcore_axis_name="s")


@functools.partial(
    pl.kernel,
    mesh=_sc_mesh,
    compiler_params=pltpu.CompilerParams(use_tc_tiling_on_sc=False),
    out_type=jax.ShapeDtypeStruct((_BATCH, _CP), jnp.float32),
    scratch_types=[
        pltpu.VMEM((_SEQ, _BPW), jnp.int32),
        pltpu.VMEM((_IPW, _CP), jnp.float32),
        pltpu.VMEM((_BPW, _CP), jnp.float32),
        pltpu.VMEM((_CP,), jnp.float32),
        pltpu.SemaphoreType.DMA,
    ],
)
def _sc_pool(xw_hbm, proj_hbm, bias_hbm, out_hbm, idx_v, rows_v, out_v, b_v, sem):
    wid = lax.axis_index("s") * _NC + lax.axis_index("c")
    pltpu.sync_copy(xw_hbm.at[wid], idx_v)
    pltpu.sync_copy(bias_hbm, b_v)

    def fire(j, carry):
        pltpu.async_copy(
            proj_hbm.at[idx_v.at[j]],
            rows_v.at[pl.ds(j * _BPW, _BPW)],
            sem,
        ).wait()
        return carry

    lax.fori_loop(0, _SEQ, fire, 0)

    def reduce_row(i, carry):
        base = i * _SEQ
        acc = b_v[...]
        for l in range(_SEQ):
            acc = acc + rows_v[base + l]
        out_v[i] = acc
        return carry

    lax.fori_loop(0, _BPW, reduce_row, 0)
    pltpu.sync_copy(out_v, out_hbm.at[pl.ds(wid * _BPW, _BPW)])


def kernel(x, emb_table, W, b):
    w_pad = jnp.pad(W, ((0, 0), (0, _CP - _CLASSES))) * (1.0 / _SEQ)
    b_pad = jnp.pad(b, (0, _CP - _CLASSES)).astype(jnp.float32)
    proj = _project_table(emb_table, w_pad)
    xw = x.astype(jnp.int32).reshape(_NW, _SEQ, _BPW)
    out = _sc_pool(xw, proj, b_pad)
    return out[:, :_CLASSES]
